# Initial kernel scaffold; baseline (speedup 1.0000x reference)
import jax
import jax.numpy as jnp
from jax.experimental import pallas as pl


def kernel(x, edge_index, edge_attr, batch, enc_W, enc_b, gat_W, gat_att_src, gat_att_dst, gat_b, gat_ln_g, gat_ln_b, gcn_W, gcn_b, gcn_ln_g, gcn_ln_b, ro_W, ro_b, fq_W, fq_b, ph_W1, ph_b1, ph_W2, ph_b2):
    raise NotImplementedError("write your pallas kernel here")


# speedup vs baseline: 11.8076x; 11.8076x over previous
import functools

import numpy as np
import jax
import jax.numpy as jnp
from jax import lax
from jax.experimental import pallas as pl
from jax.experimental.pallas import tpu as pltpu
from jax.experimental.pallas import tpu_sc as plsc

N = 50000
E = 800000
H = 64
HEADS = 8
HD = H // HEADS
WIN = 50
B = N // WIN
FREQ = WIN // 2 + 1

NSEG = 64
NP = 784
NPAD = NSEG * NP
C = 128
NCHUNK = E // C

_mesh = plsc.VectorSubcoreMesh(core_axis_name="c", subcore_axis_name="s")
_sc_params = pltpu.CompilerParams(needs_layout_passes=False, use_tc_tiling_on_sc=False)


def _wid():
    return lax.axis_index("s") * 2 + lax.axis_index("c")


def _seg_bounds(off_v, seg):
    v = off_v[pl.ds(seg, 16)]
    return v[0], v[1]


def _deg_body(dsts, ews, offs, deg_out, off_v, dv, wv, acc, cm):
    iota = lax.iota(jnp.int32, 16)
    sel0 = iota == 0
    zero16 = jnp.zeros((16,), jnp.float32)
    pltpu.sync_copy(offs, off_v)
    wid = _wid()
    for k2 in range(2):
        seg = wid * 2 + k2
        base = seg * NP
        e_lo, e_hi = _seg_bounds(off_v, seg)

        def zero_body(i, _):
            acc[pl.ds(i * 16, 16)] = zero16
            return 0
        lax.fori_loop(0, NP, zero_body, 0)

        def chunk(c, _):
            pltpu.sync_copy(dsts.at[pl.ds(c * C, C)], dv)
            pltpu.sync_copy(ews.at[pl.ds(c * C, C)], wv)
            k_lo = jnp.maximum(e_lo, c * C)
            k_hi = jnp.minimum(e_hi, (c + 1) * C)

            def edge(k, _):
                koff = k - c * C
                dl = dv[pl.ds(koff, 16)][0] - base
                w = wv[pl.ds(koff, 16)][0]
                plsc.addupdate(acc.at[pl.ds(dl * 16, 16)],
                               jnp.where(sel0, w, 0.0))
                return 0
            lax.fori_loop(k_lo, k_hi, edge, 0)
            return 0
        lax.fori_loop(e_lo // C, (e_hi + C - 1) // C, chunk, 0)

        def cp_body(i, _):
            idx = i * 256 + iota * 16
            dv_f = plsc.load_gather(acc, [idx])
            cm[pl.ds(i * 16, 16)] = dv_f + 1.0
            return 0
        lax.fori_loop(0, NP // 16, cp_body, 0)
        pltpu.sync_copy(cm, deg_out.at[seg])


def _deg_call(dsts, ews, offs):
    return pl.kernel(
        _deg_body,
        out_type=jax.ShapeDtypeStruct((NSEG, NP), jnp.float32),
        mesh=_mesh,
        compiler_params=_sc_params,
        scratch_types=[
            pltpu.VMEM((80,), jnp.int32),
            pltpu.VMEM((C,), jnp.int32),
            pltpu.VMEM((C,), jnp.float32),
            pltpu.VMEM((NP * 16,), jnp.float32),
            pltpu.VMEM((NP,), jnp.float32),
        ],
    )(dsts, ews, offs)


def _norm_body(srcs, dsts, ews, dinv, norm_out, dinv_v, sv, dv, wv, nv):
    wid = _wid()
    pltpu.sync_copy(dinv, dinv_v)
    c0 = wid * NCHUNK // 32
    c1 = (wid + 1) * NCHUNK // 32

    def chunk(c, _):
        pltpu.sync_copy(srcs.at[pl.ds(c * C, C)], sv)
        pltpu.sync_copy(dsts.at[pl.ds(c * C, C)], dv)
        pltpu.sync_copy(ews.at[pl.ds(c * C, C)], wv)
        for u in range(C // 16):
            s16 = sv[pl.ds(u * 16, 16)]
            d16 = dv[pl.ds(u * 16, 16)]
            w16 = wv[pl.ds(u * 16, 16)]
            n16 = (plsc.load_gather(dinv_v, [s16]) * w16
                   * plsc.load_gather(dinv_v, [d16]))
            nv[pl.ds(u * 16, 16)] = n16
        pltpu.sync_copy(nv, norm_out.at[pl.ds(c * C, C)])
        return 0
    lax.fori_loop(c0, c1, chunk, 0)


def _norm_call(srcs, dsts, ews, dinv):
    return pl.kernel(
        _norm_body,
        out_type=jax.ShapeDtypeStruct((E,), jnp.float32),
        mesh=_mesh,
        compiler_params=_sc_params,
        scratch_types=[
            pltpu.VMEM((NPAD,), jnp.float32),
            pltpu.VMEM((C,), jnp.int32),
            pltpu.VMEM((C,), jnp.int32),
            pltpu.VMEM((C,), jnp.float32),
            pltpu.VMEM((C,), jnp.float32),
        ],
    )(srcs, dsts, ews, dinv)


def _gcn_body(hp, srcs, dsts, norms, offs, out,
              off_v, sv, dv, nv, rows, acc, sem):
    zero16 = jnp.zeros((16,), jnp.float32)
    pltpu.sync_copy(offs, off_v)
    wid = _wid()
    for k2 in range(2):
        seg = wid * 2 + k2
        base = seg * NP
        e_lo, e_hi = _seg_bounds(off_v, seg)

        def zero_body(i, _):
            for j in range(4):
                acc[pl.ds(i * 64 + j * 16, 16)] = zero16
            return 0
        lax.fori_loop(0, NP, zero_body, 0)

        def chunk(c, _):
            pltpu.sync_copy(srcs.at[pl.ds(c * C, C)], sv)
            pltpu.sync_copy(dsts.at[pl.ds(c * C, C)], dv)
            pltpu.sync_copy(norms.at[pl.ds(c * C, C)], nv)
            pltpu.async_copy(hp.at[sv], rows, sem).wait()
            k_lo = jnp.maximum(e_lo, c * C)
            k_hi = jnp.minimum(e_hi, (c + 1) * C)

            def edge(k, _):
                koff = k - c * C
                dl = dv[pl.ds(koff, 16)][0] - base
                w = nv[pl.ds(koff, 16)][0]
                for j in range(4):
                    plsc.addupdate(acc.at[pl.ds(dl * 64 + j * 16, 16)],
                                   w * rows[koff, pl.ds(j * 16, 16)])
                return 0
            lax.fori_loop(k_lo, k_hi, edge, 0)
            return 0
        lax.fori_loop(e_lo // C, (e_hi + C - 1) // C, chunk, 0)
        pltpu.sync_copy(acc, out.at[seg])


def _gcn_call(hp, srcs, dsts, norms, offs):
    return pl.kernel(
        _gcn_body,
        out_type=jax.ShapeDtypeStruct((NSEG, NP * 64), jnp.float32),
        mesh=_mesh,
        compiler_params=_sc_params,
        scratch_types=[
            pltpu.VMEM((80,), jnp.int32),
            pltpu.VMEM((C,), jnp.int32),
            pltpu.VMEM((C,), jnp.int32),
            pltpu.VMEM((C,), jnp.float32),
            pltpu.VMEM((C, 64), jnp.float32),
            pltpu.VMEM((NP * 64,), jnp.float32),
            pltpu.SemaphoreType.DMA,
        ],
    )(hp, srcs, dsts, norms, offs)


def _gata_body(a_s, a_d_c, srcs, dsts, offs, m_out,
               off_v, sv, dv, as_ch, adv, acc_m, cmp_v, sem):
    iota = lax.iota(jnp.int32, 16)
    rsel = jnp.where(iota >= 8, 1, 0)
    ci = iota % 8
    pairsel = (iota // 8) * 16 + (iota % 8)
    neg = jnp.full((16,), -1e30, jnp.float32)
    pltpu.sync_copy(offs, off_v)
    wid = _wid()
    for k2 in range(2):
        seg = wid * 2 + k2
        base = seg * NP
        e_lo, e_hi = _seg_bounds(off_v, seg)
        pltpu.sync_copy(a_d_c.at[seg], adv.at[pl.ds(0, NP * 8)])

        def init_body(i, _):
            acc_m[pl.ds(i * 16, 16)] = neg
            return 0
        lax.fori_loop(0, NP, init_body, 0)

        def chunk(c, _):
            pltpu.sync_copy(srcs.at[pl.ds(c * C, C)], sv)
            pltpu.sync_copy(dsts.at[pl.ds(c * C, C)], dv)
            pltpu.async_copy(a_s.at[sv], as_ch.at[pl.ds(0, C)], sem).wait()
            k_lo = jnp.maximum(e_lo, c * C)
            k_hi = jnp.minimum(e_hi, (c + 1) * C)

            def edge(k, _):
                koff = k - c * C
                dl = dv[pl.ds(koff, 16)][0] - base
                as16 = plsc.load_gather(as_ch, [koff + rsel, ci])
                ad16 = adv[pl.ds(dl * 8, 16)]
                e16 = as16 + ad16
                e16 = jnp.where(e16 > 0, e16, 0.2 * e16)
                old = acc_m[pl.ds(dl * 16, 16)]
                acc_m[pl.ds(dl * 16, 16)] = jnp.maximum(old, e16)
                return 0
            lax.fori_loop(k_lo, k_hi, edge, 0)
            return 0
        lax.fori_loop(e_lo // C, (e_hi + C - 1) // C, chunk, 0)

        def cp_body(i, _):
            cmp_v[pl.ds(i * 16, 16)] = plsc.load_gather(acc_m, [i * 32 + pairsel])
            return 0
        lax.fori_loop(0, NP * 8 // 16, cp_body, 0)
        pltpu.sync_copy(cmp_v, m_out.at[seg])


def _gata_call(a_s, a_d_c, srcs, dsts, offs):
    return pl.kernel(
        _gata_body,
        out_type=jax.ShapeDtypeStruct((NSEG, NP * 8), jnp.float32),
        mesh=_mesh,
        compiler_params=_sc_params,
        scratch_types=[
            pltpu.VMEM((80,), jnp.int32),
            pltpu.VMEM((C,), jnp.int32),
            pltpu.VMEM((C,), jnp.int32),
            pltpu.VMEM((C + 2, 8), jnp.float32),
            pltpu.VMEM((NP * 8 + 16,), jnp.float32),
            pltpu.VMEM((NP * 16,), jnp.float32),
            pltpu.VMEM((NP * 8,), jnp.float32),
            pltpu.SemaphoreType.DMA,
        ],
    )(a_s, a_d_c, srcs, dsts, offs)


def _gatb_body(hp, a_s, a_d_c, m_c, srcs, dsts, offs, den_out, num_out,
               off_v, sv, dv, as_ch, adv, mv, rows, exv, acc_den, acc_num,
               cmp_v, sem):
    iota = lax.iota(jnp.int32, 16)
    rsel = jnp.where(iota >= 8, 1, 0)
    ci = iota % 8
    pairsel = (iota // 8) * 16 + (iota % 8)
    pats = [2 * j + rsel for j in range(4)]
    zero16 = jnp.zeros((16,), jnp.float32)
    pltpu.sync_copy(offs, off_v)
    wid = _wid()
    for k2 in range(2):
        seg = wid * 2 + k2
        base = seg * NP
        e_lo, e_hi = _seg_bounds(off_v, seg)
        pltpu.sync_copy(a_d_c.at[seg], adv.at[pl.ds(0, NP * 8)])
        pltpu.sync_copy(m_c.at[seg], mv.at[pl.ds(0, NP * 8)])

        def init_body(i, _):
            acc_den[pl.ds(i * 16, 16)] = zero16
            for j in range(4):
                acc_num[pl.ds(i * 64 + j * 16, 16)] = zero16
            return 0
        lax.fori_loop(0, NP, init_body, 0)

        def chunk(c, _):
            pltpu.sync_copy(srcs.at[pl.ds(c * C, C)], sv)
            pltpu.sync_copy(dsts.at[pl.ds(c * C, C)], dv)
            pltpu.async_copy(a_s.at[sv], as_ch.at[pl.ds(0, C)], sem).wait()
            pltpu.async_copy(hp.at[sv], rows, sem).wait()
            k_lo = jnp.maximum(e_lo, c * C)
            k_hi = jnp.minimum(e_hi, (c + 1) * C)

            def edge(k, _):
                koff = k - c * C
                dl = dv[pl.ds(koff, 16)][0] - base
                as16 = plsc.load_gather(as_ch, [koff + rsel, ci])
                ad16 = adv[pl.ds(dl * 8, 16)]
                e16 = as16 + ad16
                e16 = jnp.where(e16 > 0, e16, 0.2 * e16)
                m16 = mv[pl.ds(dl * 8, 16)]
                ex16 = jnp.exp(e16 - m16)
                plsc.addupdate(acc_den.at[pl.ds(dl * 16, 16)], ex16)
                exv[pl.ds(0, 16)] = ex16
                for j in range(4):
                    exj = plsc.load_gather(exv, [pats[j]])
                    plsc.addupdate(acc_num.at[pl.ds(dl * 64 + j * 16, 16)],
                                   exj * rows[koff, pl.ds(j * 16, 16)])
                return 0
            lax.fori_loop(k_lo, k_hi, edge, 0)
            return 0
        lax.fori_loop(e_lo // C, (e_hi + C - 1) // C, chunk, 0)

        def cp_body(i, _):
            cmp_v[pl.ds(i * 16, 16)] = plsc.load_gather(acc_den, [i * 32 + pairsel])
            return 0
        lax.fori_loop(0, NP * 8 // 16, cp_body, 0)
        pltpu.sync_copy(cmp_v, den_out.at[seg])
        pltpu.sync_copy(acc_num, num_out.at[seg])


def _gatb_call(hp, a_s, a_d_c, m_c, srcs, dsts, offs):
    return pl.kernel(
        _gatb_body,
        out_type=(jax.ShapeDtypeStruct((NSEG, NP * 8), jnp.float32),
                  jax.ShapeDtypeStruct((NSEG, NP * 64), jnp.float32)),
        mesh=_mesh,
        compiler_params=_sc_params,
        scratch_types=[
            pltpu.VMEM((80,), jnp.int32),
            pltpu.VMEM((C,), jnp.int32),
            pltpu.VMEM((C,), jnp.int32),
            pltpu.VMEM((C + 2, 8), jnp.float32),
            pltpu.VMEM((NP * 8 + 16,), jnp.float32),
            pltpu.VMEM((NP * 8 + 16,), jnp.float32),
            pltpu.VMEM((C, 64), jnp.float32),
            pltpu.VMEM((16,), jnp.float32),
            pltpu.VMEM((NP * 16,), jnp.float32),
            pltpu.VMEM((NP * 64,), jnp.float32),
            pltpu.VMEM((NP * 8,), jnp.float32),
            pltpu.SemaphoreType.DMA,
        ],
    )(hp, a_s, a_d_c, m_c, srcs, dsts, offs)


def _gelu(x):
    return 0.5 * x * (1.0 + lax.erf(x * 0.7071067811865476))


def _ln(x, g, b):
    mu = jnp.mean(x, axis=-1, keepdims=True)
    v = jnp.mean((x - mu) ** 2, axis=-1, keepdims=True)
    return (x - mu) / jnp.sqrt(v + 1e-5) * g + b


ROWB = 512
GRID = NPAD // ROWB


def _t1_body(x_ref, ew_ref, eb_ref, gw_ref, as_ref, ad_ref,
             hp_ref, s_ref, d_ref):
    h0 = _gelu(jnp.dot(x_ref[...], ew_ref[...],
                       preferred_element_type=jnp.float32) + eb_ref[...])
    hp = jnp.dot(h0, gw_ref[...], preferred_element_type=jnp.float32)
    hp_ref[...] = hp
    s_ref[...] = jnp.dot(hp, as_ref[...], preferred_element_type=jnp.float32)
    d_ref[...] = jnp.dot(hp, ad_ref[...], preferred_element_type=jnp.float32)


def _t1_call(xp, enc_W, enc_b, gat_W, As, Ad):
    return pl.pallas_call(
        _t1_body,
        grid=(GRID,),
        in_specs=[
            pl.BlockSpec((ROWB, 4), lambda i: (i, 0)),
            pl.BlockSpec((4, H), lambda i: (0, 0)),
            pl.BlockSpec((1, H), lambda i: (0, 0)),
            pl.BlockSpec((H, H), lambda i: (0, 0)),
            pl.BlockSpec((H, 8), lambda i: (0, 0)),
            pl.BlockSpec((H, 8), lambda i: (0, 0)),
        ],
        out_specs=[
            pl.BlockSpec((ROWB, H), lambda i: (i, 0)),
            pl.BlockSpec((ROWB, 8), lambda i: (i, 0)),
            pl.BlockSpec((ROWB, 8), lambda i: (i, 0)),
        ],
        out_shape=[
            jax.ShapeDtypeStruct((NPAD, H), jnp.float32),
            jax.ShapeDtypeStruct((NPAD, 8), jnp.float32),
            jax.ShapeDtypeStruct((NPAD, 8), jnp.float32),
        ],
    )(xp, enc_W, enc_b.reshape(1, H), gat_W, As, Ad)


def _t6_body(deg_ref, dinv_ref):
    dinv_ref[...] = lax.rsqrt(deg_ref[...])


def _t6_call(deg):
    return pl.pallas_call(
        _t6_body,
        out_shape=jax.ShapeDtypeStruct((392, 128), jnp.float32),
    )(deg.reshape(392, 128))


def _t2_body(me_ref, as_ref, ad_ref, m_ref):
    es = as_ref[...] + ad_ref[...]
    es = jnp.where(es > 0, es, 0.2 * es)
    m_ref[...] = jnp.maximum(me_ref[...], es)


def _t2_call(m_e, a_s, a_d):
    return pl.pallas_call(
        _t2_body,
        grid=(GRID,),
        in_specs=[pl.BlockSpec((ROWB, 8), lambda i: (i, 0))] * 3,
        out_specs=pl.BlockSpec((ROWB, 8), lambda i: (i, 0)),
        out_shape=jax.ShapeDtypeStruct((NPAD, 8), jnp.float32),
    )(m_e, a_s, a_d)


def _t3_body(num_ref, den_ref, m_ref, as_ref, ad_ref, hp_ref, r_ref,
             b_ref, g_ref, lb_ref, h_ref):
    es = as_ref[...] + ad_ref[...]
    es = jnp.where(es > 0, es, 0.2 * es)
    w = jnp.exp(es - m_ref[...])
    den = den_ref[...] + w + 1e-16
    w64 = jnp.dot(w, r_ref[...], preferred_element_type=jnp.float32)
    den64 = jnp.dot(den, r_ref[...], preferred_element_type=jnp.float32)
    out = (num_ref[...] + w64 * hp_ref[...]) / den64 + b_ref[...]
    h_ref[...] = _ln(out, g_ref[...], lb_ref[...])


def _t3_call(num, den, m, a_s, a_d, hp, R, gat_b, ln_g, ln_b):
    return pl.pallas_call(
        _t3_body,
        grid=(GRID,),
        in_specs=[
            pl.BlockSpec((ROWB, H), lambda i: (i, 0)),
            pl.BlockSpec((ROWB, 8), lambda i: (i, 0)),
            pl.BlockSpec((ROWB, 8), lambda i: (i, 0)),
            pl.BlockSpec((ROWB, 8), lambda i: (i, 0)),
            pl.BlockSpec((ROWB, 8), lambda i: (i, 0)),
            pl.BlockSpec((ROWB, H), lambda i: (i, 0)),
            pl.BlockSpec((8, H), lambda i: (0, 0)),
            pl.BlockSpec((1, H), lambda i: (0, 0)),
            pl.BlockSpec((1, H), lambda i: (0, 0)),
            pl.BlockSpec((1, H), lambda i: (0, 0)),
        ],
        out_specs=pl.BlockSpec((ROWB, H), lambda i: (i, 0)),
        out_shape=jax.ShapeDtypeStruct((NPAD, H), jnp.float32),
    )(num, den, m, a_s, a_d, hp, R, gat_b.reshape(1, H),
      ln_g.reshape(1, H), ln_b.reshape(1, H))


def _t4_body(h_ref, w_ref, o_ref):
    o_ref[...] = jnp.dot(h_ref[...], w_ref[...],
                         preferred_element_type=jnp.float32)


def _t4_call(h, W):
    return pl.pallas_call(
        _t4_body,
        grid=(GRID,),
        in_specs=[
            pl.BlockSpec((ROWB, H), lambda i: (i, 0)),
            pl.BlockSpec((H, H), lambda i: (0, 0)),
        ],
        out_specs=pl.BlockSpec((ROWB, H), lambda i: (i, 0)),
        out_shape=jax.ShapeDtypeStruct((NPAD, H), jnp.float32),
    )(h, W)


def _t5_body(s_ref, hp_ref, deg_ref, b_ref, g_ref, lb_ref, res_ref, o_ref):
    t = s_ref[...] + hp_ref[...] / deg_ref[...] + b_ref[...]
    o = _gelu(_ln(t, g_ref[...], lb_ref[...]))
    if res_ref is not None:
        o = o + res_ref[...]
    o_ref[...] = o


def _t5_call(S, hp, deg_col, b, g, lb, res=None):
    ins = [S, hp, deg_col, b.reshape(1, H), g.reshape(1, H), lb.reshape(1, H)]
    specs = [
        pl.BlockSpec((ROWB, H), lambda i: (i, 0)),
        pl.BlockSpec((ROWB, H), lambda i: (i, 0)),
        pl.BlockSpec((ROWB, 1), lambda i: (i, 0)),
        pl.BlockSpec((1, H), lambda i: (0, 0)),
        pl.BlockSpec((1, H), lambda i: (0, 0)),
        pl.BlockSpec((1, H), lambda i: (0, 0)),
    ]
    if res is not None:
        ins.append(res)
        specs.append(pl.BlockSpec((ROWB, H), lambda i: (i, 0)))
        body = _t5_body
    else:
        def body(s_ref, hp_ref, deg_ref, b_ref, g_ref, lb_ref, o_ref):
            _t5_body(s_ref, hp_ref, deg_ref, b_ref, g_ref, lb_ref, None, o_ref)
    return pl.pallas_call(
        body,
        grid=(GRID,),
        in_specs=specs,
        out_specs=pl.BlockSpec((ROWB, H), lambda i: (i, 0)),
        out_shape=jax.ShapeDtypeStruct((NPAD, H), jnp.float32),
    )(*ins)


def _t7_body(h3_ref, ts_ref, cm_ref, sm_ref, row_ref, rob_ref,
             fqw_ref, fqb_ref, w1a_ref, w1b_ref, b1_ref, w2_ref, b2_ref,
             o_ref):
    gsum = jnp.sum(h3_ref[...], axis=1)
    gemb = gsum * (1.0 / WIN)
    gfeat = _gelu(jnp.dot(gemb, row_ref[...],
                          preferred_element_type=jnp.float32) + rob_ref[...])
    ts = ts_ref[...]
    fr = jnp.dot(ts, cm_ref[...], preferred_element_type=jnp.float32)
    fi = jnp.dot(ts, sm_ref[...], preferred_element_type=jnp.float32)
    ff = jnp.sqrt(fr * fr + fi * fi)
    ffeat = _gelu(jnp.dot(ff, fqw_ref[...],
                          preferred_element_type=jnp.float32) + fqb_ref[...])
    hidden = _gelu(jnp.dot(gfeat, w1a_ref[...],
                           preferred_element_type=jnp.float32)
                   + jnp.dot(ffeat, w1b_ref[...],
                             preferred_element_type=jnp.float32)
                   + b1_ref[...])
    o_ref[...] = jnp.dot(hidden, w2_ref[...],
                         preferred_element_type=jnp.float32) + b2_ref[...]


def _t7_call(h3, ts, Cm, Sm, ro_W, ro_b, fq_W, fq_b, W1a, W1b, b1, W2, b2):
    return pl.pallas_call(
        _t7_body,
        out_shape=jax.ShapeDtypeStruct((B, 1), jnp.float32),
    )(h3, ts, Cm, Sm, ro_W, ro_b.reshape(1, H), fq_W, fq_b.reshape(1, H),
      W1a, W1b, b1.reshape(1, H), W2, b2.reshape(1, 1))


def kernel(x, edge_index, edge_attr, batch, enc_W, enc_b, gat_W, gat_att_src,
           gat_att_dst, gat_b, gat_ln_g, gat_ln_b, gcn_W, gcn_b, gcn_ln_g,
           gcn_ln_b, ro_W, ro_b, fq_W, fq_b, ph_W1, ph_b1, ph_W2, ph_b2):
    src = edge_index[0]
    dst = edge_index[1]
    ew = edge_attr[:, 0]

    order = jnp.argsort(dst)
    dsts = jnp.take(dst, order)
    srcs = jnp.take(src, order)
    ews = jnp.take(ew, order)
    bounds = jnp.arange(NSEG + 1, dtype=jnp.int32) * NP
    offs = jnp.zeros((80,), jnp.int32).at[: NSEG + 1].set(
        jnp.searchsorted(dsts, bounds).astype(jnp.int32))

    idx64 = np.arange(H)
    As = jnp.zeros((H, 8), jnp.float32).at[idx64, idx64 // 8].set(
        gat_att_src.reshape(-1))
    Ad = jnp.zeros((H, 8), jnp.float32).at[idx64, idx64 // 8].set(
        gat_att_dst.reshape(-1))
    R = jnp.zeros((8, H), jnp.float32).at[idx64 // 8, idx64].set(1.0)
    nk = np.arange(WIN)[:, None] * np.arange(FREQ)[None, :]
    ang = 2.0 * np.pi * nk / WIN
    Cm = jnp.asarray(np.cos(ang), dtype=jnp.float32)
    Sm = jnp.asarray(np.sin(ang), dtype=jnp.float32)

    xp = jnp.pad(x, ((0, NPAD - N), (0, 0)))

    hp_g, a_s, a_d = _t1_call(xp, enc_W, enc_b, gat_W, As, Ad)

    deg_c = _deg_call(dsts, ews, offs)
    deg_flat = deg_c.reshape(NPAD)
    dinv = _t6_call(deg_flat).reshape(NPAD)
    norms = _norm_call(srcs, dsts, ews, dinv)

    a_d_c = a_d.reshape(NSEG, NP * 8)
    m_e = _gata_call(a_s, a_d_c, srcs, dsts, offs)
    m = _t2_call(m_e.reshape(NPAD, 8), a_s, a_d)
    den_c, num_c = _gatb_call(hp_g, a_s, a_d_c, m.reshape(NSEG, NP * 8),
                              srcs, dsts, offs)
    h = _t3_call(num_c.reshape(NPAD, H), den_c.reshape(NPAD, 8), m,
                 a_s, a_d, hp_g, R, gat_b, gat_ln_g, gat_ln_b)

    deg_col = deg_flat.reshape(NPAD, 1)
    for blk in range(4):
        res = h
        hp1 = _t4_call(h, gcn_W[2 * blk])
        S1 = _gcn_call(hp1, srcs, dsts, norms, offs).reshape(NPAD, H)
        o1 = _t5_call(S1, hp1, deg_col, gcn_b[2 * blk],
                      gcn_ln_g[2 * blk], gcn_ln_b[2 * blk])
        hp2 = _t4_call(o1, gcn_W[2 * blk + 1])
        S2 = _gcn_call(hp2, srcs, dsts, norms, offs).reshape(NPAD, H)
        h = _t5_call(S2, hp2, deg_col, gcn_b[2 * blk + 1],
                     gcn_ln_g[2 * blk + 1], gcn_ln_b[2 * blk + 1], res=res)

    h3 = h[:N].reshape(B, WIN, H)
    ts = x[:, 0].reshape(B, WIN)
    out = _t7_call(h3, ts, Cm, Sm, ro_W, ro_b, fq_W, fq_b,
                   ph_W1[:H], ph_W1[H:], ph_b1, ph_W2, ph_b2)
    return out


# speedup vs baseline: 12.4517x; 1.0545x over previous
import functools

import numpy as np
import jax
import jax.numpy as jnp
from jax import lax
from jax.experimental import pallas as pl
from jax.experimental.pallas import tpu as pltpu
from jax.experimental.pallas import tpu_sc as plsc

N = 50000
E = 800000
H = 64
HEADS = 8
HD = H // HEADS
WIN = 50
B = N // WIN
FREQ = WIN // 2 + 1

NSEG = 64
NP = 784
NPAD = NSEG * NP
C = 128
NCHUNK = E // C

_mesh = plsc.VectorSubcoreMesh(core_axis_name="c", subcore_axis_name="s")
_sc_params = pltpu.CompilerParams(needs_layout_passes=False, use_tc_tiling_on_sc=False)


def _wid():
    return lax.axis_index("s") * 2 + lax.axis_index("c")


def _seg_bounds(off_v, seg):
    v = off_v[pl.ds(seg, 16)]
    return v[0], v[1]


def _deg_body(dsts, ews, offs, deg_out, off_v, dv, wv, acc, cm):
    iota = lax.iota(jnp.int32, 16)
    sel0 = iota == 0
    zero16 = jnp.zeros((16,), jnp.float32)
    pltpu.sync_copy(offs, off_v)
    wid = _wid()
    for k2 in range(2):
        seg = wid * 2 + k2
        base = seg * NP
        e_lo, e_hi = _seg_bounds(off_v, seg)

        def zero_body(i, _):
            acc[pl.ds(i * 16, 16)] = zero16
            return 0
        lax.fori_loop(0, NP, zero_body, 0)

        def chunk(c, _):
            pltpu.sync_copy(dsts.at[pl.ds(c * C, C)], dv)
            pltpu.sync_copy(ews.at[pl.ds(c * C, C)], wv)
            k_lo = jnp.maximum(e_lo, c * C)
            k_hi = jnp.minimum(e_hi, (c + 1) * C)

            def edge(k, _):
                koff = k - c * C
                dl = dv[pl.ds(koff, 16)][0] - base
                w = wv[pl.ds(koff, 16)][0]
                plsc.addupdate(acc.at[pl.ds(dl * 16, 16)],
                               jnp.where(sel0, w, 0.0))
                return 0
            lax.fori_loop(k_lo, k_hi, edge, 0)
            return 0
        lax.fori_loop(e_lo // C, (e_hi + C - 1) // C, chunk, 0)

        def cp_body(i, _):
            idx = i * 256 + iota * 16
            dv_f = plsc.load_gather(acc, [idx])
            cm[pl.ds(i * 16, 16)] = dv_f + 1.0
            return 0
        lax.fori_loop(0, NP // 16, cp_body, 0)
        pltpu.sync_copy(cm, deg_out.at[seg])


def _deg_call(dsts, ews, offs):
    return pl.kernel(
        _deg_body,
        out_type=jax.ShapeDtypeStruct((NSEG, NP), jnp.float32),
        mesh=_mesh,
        compiler_params=_sc_params,
        scratch_types=[
            pltpu.VMEM((80,), jnp.int32),
            pltpu.VMEM((C,), jnp.int32),
            pltpu.VMEM((C,), jnp.float32),
            pltpu.VMEM((NP * 16,), jnp.float32),
            pltpu.VMEM((NP,), jnp.float32),
        ],
    )(dsts, ews, offs)


def _norm_body(srcs, dsts, ews, dinv, norm_out, dinv_v, sv, dv, wv, nv):
    wid = _wid()
    pltpu.sync_copy(dinv, dinv_v)
    c0 = wid * NCHUNK // 32
    c1 = (wid + 1) * NCHUNK // 32

    def chunk(c, _):
        pltpu.sync_copy(srcs.at[pl.ds(c * C, C)], sv)
        pltpu.sync_copy(dsts.at[pl.ds(c * C, C)], dv)
        pltpu.sync_copy(ews.at[pl.ds(c * C, C)], wv)
        for u in range(C // 16):
            s16 = sv[pl.ds(u * 16, 16)]
            d16 = dv[pl.ds(u * 16, 16)]
            w16 = wv[pl.ds(u * 16, 16)]
            n16 = (plsc.load_gather(dinv_v, [s16]) * w16
                   * plsc.load_gather(dinv_v, [d16]))
            nv[pl.ds(u * 16, 16)] = n16
        pltpu.sync_copy(nv, norm_out.at[pl.ds(c * C, C)])
        return 0
    lax.fori_loop(c0, c1, chunk, 0)


def _norm_call(srcs, dsts, ews, dinv):
    return pl.kernel(
        _norm_body,
        out_type=jax.ShapeDtypeStruct((E,), jnp.float32),
        mesh=_mesh,
        compiler_params=_sc_params,
        scratch_types=[
            pltpu.VMEM((NPAD,), jnp.float32),
            pltpu.VMEM((C,), jnp.int32),
            pltpu.VMEM((C,), jnp.int32),
            pltpu.VMEM((C,), jnp.float32),
            pltpu.VMEM((C,), jnp.float32),
        ],
    )(srcs, dsts, ews, dinv)


def _gcn_body(hp, srcs, dsts, norms, offs, out,
              off_v, sv, dv, nv, rows, acc, sem):
    iota = lax.iota(jnp.int32, 16)
    zero16 = jnp.zeros((16,), jnp.float32)
    pltpu.sync_copy(offs, off_v)
    wid = _wid()
    for k2 in range(2):
        seg = wid * 2 + k2
        base = seg * NP
        e_lo, e_hi = _seg_bounds(off_v, seg)

        def zero_body(i, _):
            for j in range(4):
                acc[pl.ds(i * 64 + j * 16, 16)] = zero16
            return 0
        lax.fori_loop(0, NP, zero_body, 0)

        def chunk(c, _):
            pltpu.sync_copy(srcs.at[pl.ds(c * C, C)], sv)
            pltpu.sync_copy(dsts.at[pl.ds(c * C, C)], dv)
            pltpu.sync_copy(norms.at[pl.ds(c * C, C)], nv.at[pl.ds(0, C)])
            pltpu.async_copy(hp.at[sv], rows, sem).wait()
            g_lo = e_lo - c * C
            g_hi = e_hi - c * C
            for u in range(8):
                lidx = iota + u * 16
                valid = (lidx >= g_lo) & (lidx < g_hi)
                n16 = jnp.where(valid, nv[pl.ds(u * 16, 16)], 0.0)
                dl16 = jnp.clip(dv[pl.ds(u * 16, 16)] - base, 0, NP - 1)
                nv[pl.ds(u * 16, 16)] = n16
                for i in range(16):
                    dl = dl16[i]
                    w = nv[pl.ds(u * 16 + i, 16)][0]
                    for j in range(4):
                        plsc.addupdate(
                            acc.at[pl.ds(dl * 64 + j * 16, 16)],
                            w * rows[u * 16 + i, pl.ds(j * 16, 16)])
            return 0
        lax.fori_loop(e_lo // C, (e_hi + C - 1) // C, chunk, 0)
        pltpu.sync_copy(acc, out.at[seg])


def _gcn_call(hp, srcs, dsts, norms, offs):
    return pl.kernel(
        _gcn_body,
        out_type=jax.ShapeDtypeStruct((NSEG, NP * 64), jnp.float32),
        mesh=_mesh,
        compiler_params=_sc_params,
        scratch_types=[
            pltpu.VMEM((80,), jnp.int32),
            pltpu.VMEM((C,), jnp.int32),
            pltpu.VMEM((C,), jnp.int32),
            pltpu.VMEM((C + 16,), jnp.float32),
            pltpu.VMEM((C, 64), jnp.float32),
            pltpu.VMEM((NP * 64,), jnp.float32),
            pltpu.SemaphoreType.DMA,
        ],
    )(hp, srcs, dsts, norms, offs)


def _gata_body(a_s, a_d_c, srcs, dsts, offs, m_out,
               off_v, sv, dv, as_ch, adv, acc_m, cmp_v, sem):
    iota = lax.iota(jnp.int32, 16)
    rsel = jnp.where(iota >= 8, 1, 0)
    ci = iota % 8
    pairsel = (iota // 8) * 16 + (iota % 8)
    neg = jnp.full((16,), -1e30, jnp.float32)
    pltpu.sync_copy(offs, off_v)
    wid = _wid()
    for k2 in range(2):
        seg = wid * 2 + k2
        base = seg * NP
        e_lo, e_hi = _seg_bounds(off_v, seg)
        pltpu.sync_copy(a_d_c.at[seg], adv.at[pl.ds(0, NP * 8)])

        def init_body(i, _):
            acc_m[pl.ds(i * 16, 16)] = neg
            return 0
        lax.fori_loop(0, NP, init_body, 0)

        def chunk(c, _):
            pltpu.sync_copy(srcs.at[pl.ds(c * C, C)], sv)
            pltpu.sync_copy(dsts.at[pl.ds(c * C, C)], dv)
            pltpu.async_copy(a_s.at[sv], as_ch.at[pl.ds(0, C)], sem).wait()
            k_lo = jnp.maximum(e_lo, c * C)
            k_hi = jnp.minimum(e_hi, (c + 1) * C)

            def edge(k, _):
                koff = k - c * C
                dl = dv[pl.ds(koff, 16)][0] - base
                as16 = plsc.load_gather(as_ch, [koff + rsel, ci])
                ad16 = adv[pl.ds(dl * 8, 16)]
                e16 = as16 + ad16
                e16 = jnp.where(e16 > 0, e16, 0.2 * e16)
                old = acc_m[pl.ds(dl * 16, 16)]
                acc_m[pl.ds(dl * 16, 16)] = jnp.maximum(old, e16)
                return 0
            lax.fori_loop(k_lo, k_hi, edge, 0)
            return 0
        lax.fori_loop(e_lo // C, (e_hi + C - 1) // C, chunk, 0)

        def cp_body(i, _):
            cmp_v[pl.ds(i * 16, 16)] = plsc.load_gather(acc_m, [i * 32 + pairsel])
            return 0
        lax.fori_loop(0, NP * 8 // 16, cp_body, 0)
        pltpu.sync_copy(cmp_v, m_out.at[seg])


def _gata_call(a_s, a_d_c, srcs, dsts, offs):
    return pl.kernel(
        _gata_body,
        out_type=jax.ShapeDtypeStruct((NSEG, NP * 8), jnp.float32),
        mesh=_mesh,
        compiler_params=_sc_params,
        scratch_types=[
            pltpu.VMEM((80,), jnp.int32),
            pltpu.VMEM((C,), jnp.int32),
            pltpu.VMEM((C,), jnp.int32),
            pltpu.VMEM((C + 2, 8), jnp.float32),
            pltpu.VMEM((NP * 8 + 16,), jnp.float32),
            pltpu.VMEM((NP * 16,), jnp.float32),
            pltpu.VMEM((NP * 8,), jnp.float32),
            pltpu.SemaphoreType.DMA,
        ],
    )(a_s, a_d_c, srcs, dsts, offs)


def _gatb_body(hp, a_s, a_d_c, m_c, srcs, dsts, offs, den_out, num_out,
               off_v, sv, dv, as_ch, adv, mv, rows, exv, acc_den, acc_num,
               cmp_v, sem):
    iota = lax.iota(jnp.int32, 16)
    rsel = jnp.where(iota >= 8, 1, 0)
    ci = iota % 8
    pairsel = (iota // 8) * 16 + (iota % 8)
    pats = [2 * j + rsel for j in range(4)]
    zero16 = jnp.zeros((16,), jnp.float32)
    pltpu.sync_copy(offs, off_v)
    wid = _wid()
    for k2 in range(2):
        seg = wid * 2 + k2
        base = seg * NP
        e_lo, e_hi = _seg_bounds(off_v, seg)
        pltpu.sync_copy(a_d_c.at[seg], adv.at[pl.ds(0, NP * 8)])
        pltpu.sync_copy(m_c.at[seg], mv.at[pl.ds(0, NP * 8)])

        def init_body(i, _):
            acc_den[pl.ds(i * 16, 16)] = zero16
            for j in range(4):
                acc_num[pl.ds(i * 64 + j * 16, 16)] = zero16
            return 0
        lax.fori_loop(0, NP, init_body, 0)

        def chunk(c, _):
            pltpu.sync_copy(srcs.at[pl.ds(c * C, C)], sv)
            pltpu.sync_copy(dsts.at[pl.ds(c * C, C)], dv)
            pltpu.async_copy(a_s.at[sv], as_ch.at[pl.ds(0, C)], sem).wait()
            pltpu.async_copy(hp.at[sv], rows, sem).wait()
            k_lo = jnp.maximum(e_lo, c * C)
            k_hi = jnp.minimum(e_hi, (c + 1) * C)

            def edge(k, _):
                koff = k - c * C
                dl = dv[pl.ds(koff, 16)][0] - base
                as16 = plsc.load_gather(as_ch, [koff + rsel, ci])
                ad16 = adv[pl.ds(dl * 8, 16)]
                e16 = as16 + ad16
                e16 = jnp.where(e16 > 0, e16, 0.2 * e16)
                m16 = mv[pl.ds(dl * 8, 16)]
                ex16 = jnp.exp(e16 - m16)
                plsc.addupdate(acc_den.at[pl.ds(dl * 16, 16)], ex16)
                exv[pl.ds(0, 16)] = ex16
                for j in range(4):
                    exj = plsc.load_gather(exv, [pats[j]])
                    plsc.addupdate(acc_num.at[pl.ds(dl * 64 + j * 16, 16)],
                                   exj * rows[koff, pl.ds(j * 16, 16)])
                return 0
            lax.fori_loop(k_lo, k_hi, edge, 0)
            return 0
        lax.fori_loop(e_lo // C, (e_hi + C - 1) // C, chunk, 0)

        def cp_body(i, _):
            cmp_v[pl.ds(i * 16, 16)] = plsc.load_gather(acc_den, [i * 32 + pairsel])
            return 0
        lax.fori_loop(0, NP * 8 // 16, cp_body, 0)
        pltpu.sync_copy(cmp_v, den_out.at[seg])
        pltpu.sync_copy(acc_num, num_out.at[seg])


def _gatb_call(hp, a_s, a_d_c, m_c, srcs, dsts, offs):
    return pl.kernel(
        _gatb_body,
        out_type=(jax.ShapeDtypeStruct((NSEG, NP * 8), jnp.float32),
                  jax.ShapeDtypeStruct((NSEG, NP * 64), jnp.float32)),
        mesh=_mesh,
        compiler_params=_sc_params,
        scratch_types=[
            pltpu.VMEM((80,), jnp.int32),
            pltpu.VMEM((C,), jnp.int32),
            pltpu.VMEM((C,), jnp.int32),
            pltpu.VMEM((C + 2, 8), jnp.float32),
            pltpu.VMEM((NP * 8 + 16,), jnp.float32),
            pltpu.VMEM((NP * 8 + 16,), jnp.float32),
            pltpu.VMEM((C, 64), jnp.float32),
            pltpu.VMEM((16,), jnp.float32),
            pltpu.VMEM((NP * 16,), jnp.float32),
            pltpu.VMEM((NP * 64,), jnp.float32),
            pltpu.VMEM((NP * 8,), jnp.float32),
            pltpu.SemaphoreType.DMA,
        ],
    )(hp, a_s, a_d_c, m_c, srcs, dsts, offs)


def _gelu(x):
    return 0.5 * x * (1.0 + lax.erf(x * 0.7071067811865476))


def _ln(x, g, b):
    mu = jnp.mean(x, axis=-1, keepdims=True)
    v = jnp.mean((x - mu) ** 2, axis=-1, keepdims=True)
    return (x - mu) / jnp.sqrt(v + 1e-5) * g + b


ROWB = 512
GRID = NPAD // ROWB


def _t1_body(x_ref, ew_ref, eb_ref, gw_ref, as_ref, ad_ref,
             hp_ref, s_ref, d_ref):
    h0 = _gelu(jnp.dot(x_ref[...], ew_ref[...],
                       preferred_element_type=jnp.float32) + eb_ref[...])
    hp = jnp.dot(h0, gw_ref[...], preferred_element_type=jnp.float32)
    hp_ref[...] = hp
    s_ref[...] = jnp.dot(hp, as_ref[...], preferred_element_type=jnp.float32)
    d_ref[...] = jnp.dot(hp, ad_ref[...], preferred_element_type=jnp.float32)


def _t1_call(xp, enc_W, enc_b, gat_W, As, Ad):
    return pl.pallas_call(
        _t1_body,
        grid=(GRID,),
        in_specs=[
            pl.BlockSpec((ROWB, 4), lambda i: (i, 0)),
            pl.BlockSpec((4, H), lambda i: (0, 0)),
            pl.BlockSpec((1, H), lambda i: (0, 0)),
            pl.BlockSpec((H, H), lambda i: (0, 0)),
            pl.BlockSpec((H, 8), lambda i: (0, 0)),
            pl.BlockSpec((H, 8), lambda i: (0, 0)),
        ],
        out_specs=[
            pl.BlockSpec((ROWB, H), lambda i: (i, 0)),
            pl.BlockSpec((ROWB, 8), lambda i: (i, 0)),
            pl.BlockSpec((ROWB, 8), lambda i: (i, 0)),
        ],
        out_shape=[
            jax.ShapeDtypeStruct((NPAD, H), jnp.float32),
            jax.ShapeDtypeStruct((NPAD, 8), jnp.float32),
            jax.ShapeDtypeStruct((NPAD, 8), jnp.float32),
        ],
    )(xp, enc_W, enc_b.reshape(1, H), gat_W, As, Ad)


def _t6_body(deg_ref, dinv_ref):
    dinv_ref[...] = lax.rsqrt(deg_ref[...])


def _t6_call(deg):
    return pl.pallas_call(
        _t6_body,
        out_shape=jax.ShapeDtypeStruct((392, 128), jnp.float32),
    )(deg.reshape(392, 128))


def _t2_body(me_ref, as_ref, ad_ref, m_ref):
    es = as_ref[...] + ad_ref[...]
    es = jnp.where(es > 0, es, 0.2 * es)
    m_ref[...] = jnp.maximum(me_ref[...], es)


def _t2_call(m_e, a_s, a_d):
    return pl.pallas_call(
        _t2_body,
        grid=(GRID,),
        in_specs=[pl.BlockSpec((ROWB, 8), lambda i: (i, 0))] * 3,
        out_specs=pl.BlockSpec((ROWB, 8), lambda i: (i, 0)),
        out_shape=jax.ShapeDtypeStruct((NPAD, 8), jnp.float32),
    )(m_e, a_s, a_d)


def _t3_body(num_ref, den_ref, m_ref, as_ref, ad_ref, hp_ref, r_ref,
             b_ref, g_ref, lb_ref, h_ref):
    es = as_ref[...] + ad_ref[...]
    es = jnp.where(es > 0, es, 0.2 * es)
    w = jnp.exp(es - m_ref[...])
    den = den_ref[...] + w + 1e-16
    w64 = jnp.dot(w, r_ref[...], preferred_element_type=jnp.float32)
    den64 = jnp.dot(den, r_ref[...], preferred_element_type=jnp.float32)
    out = (num_ref[...] + w64 * hp_ref[...]) / den64 + b_ref[...]
    h_ref[...] = _ln(out, g_ref[...], lb_ref[...])


def _t3_call(num, den, m, a_s, a_d, hp, R, gat_b, ln_g, ln_b):
    return pl.pallas_call(
        _t3_body,
        grid=(GRID,),
        in_specs=[
            pl.BlockSpec((ROWB, H), lambda i: (i, 0)),
            pl.BlockSpec((ROWB, 8), lambda i: (i, 0)),
            pl.BlockSpec((ROWB, 8), lambda i: (i, 0)),
            pl.BlockSpec((ROWB, 8), lambda i: (i, 0)),
            pl.BlockSpec((ROWB, 8), lambda i: (i, 0)),
            pl.BlockSpec((ROWB, H), lambda i: (i, 0)),
            pl.BlockSpec((8, H), lambda i: (0, 0)),
            pl.BlockSpec((1, H), lambda i: (0, 0)),
            pl.BlockSpec((1, H), lambda i: (0, 0)),
            pl.BlockSpec((1, H), lambda i: (0, 0)),
        ],
        out_specs=pl.BlockSpec((ROWB, H), lambda i: (i, 0)),
        out_shape=jax.ShapeDtypeStruct((NPAD, H), jnp.float32),
    )(num, den, m, a_s, a_d, hp, R, gat_b.reshape(1, H),
      ln_g.reshape(1, H), ln_b.reshape(1, H))


def _t4_body(h_ref, w_ref, o_ref):
    o_ref[...] = jnp.dot(h_ref[...], w_ref[...],
                         preferred_element_type=jnp.float32)


def _t4_call(h, W):
    return pl.pallas_call(
        _t4_body,
        grid=(GRID,),
        in_specs=[
            pl.BlockSpec((ROWB, H), lambda i: (i, 0)),
            pl.BlockSpec((H, H), lambda i: (0, 0)),
        ],
        out_specs=pl.BlockSpec((ROWB, H), lambda i: (i, 0)),
        out_shape=jax.ShapeDtypeStruct((NPAD, H), jnp.float32),
    )(h, W)


def _t5_body(s_ref, hp_ref, deg_ref, b_ref, g_ref, lb_ref, res_ref, o_ref):
    t = s_ref[...] + hp_ref[...] / deg_ref[...] + b_ref[...]
    o = _gelu(_ln(t, g_ref[...], lb_ref[...]))
    if res_ref is not None:
        o = o + res_ref[...]
    o_ref[...] = o


def _t5_call(S, hp, deg_col, b, g, lb, res=None):
    ins = [S, hp, deg_col, b.reshape(1, H), g.reshape(1, H), lb.reshape(1, H)]
    specs = [
        pl.BlockSpec((ROWB, H), lambda i: (i, 0)),
        pl.BlockSpec((ROWB, H), lambda i: (i, 0)),
        pl.BlockSpec((ROWB, 1), lambda i: (i, 0)),
        pl.BlockSpec((1, H), lambda i: (0, 0)),
        pl.BlockSpec((1, H), lambda i: (0, 0)),
        pl.BlockSpec((1, H), lambda i: (0, 0)),
    ]
    if res is not None:
        ins.append(res)
        specs.append(pl.BlockSpec((ROWB, H), lambda i: (i, 0)))
        body = _t5_body
    else:
        def body(s_ref, hp_ref, deg_ref, b_ref, g_ref, lb_ref, o_ref):
            _t5_body(s_ref, hp_ref, deg_ref, b_ref, g_ref, lb_ref, None, o_ref)
    return pl.pallas_call(
        body,
        grid=(GRID,),
        in_specs=specs,
        out_specs=pl.BlockSpec((ROWB, H), lambda i: (i, 0)),
        out_shape=jax.ShapeDtypeStruct((NPAD, H), jnp.float32),
    )(*ins)


def _t7_body(h3_ref, ts_ref, cm_ref, sm_ref, row_ref, rob_ref,
             fqw_ref, fqb_ref, w1a_ref, w1b_ref, b1_ref, w2_ref, b2_ref,
             o_ref):
    gsum = jnp.sum(h3_ref[...], axis=1)
    gemb = gsum * (1.0 / WIN)
    gfeat = _gelu(jnp.dot(gemb, row_ref[...],
                          preferred_element_type=jnp.float32) + rob_ref[...])
    ts = ts_ref[...]
    fr = jnp.dot(ts, cm_ref[...], preferred_element_type=jnp.float32)
    fi = jnp.dot(ts, sm_ref[...], preferred_element_type=jnp.float32)
    ff = jnp.sqrt(fr * fr + fi * fi)
    ffeat = _gelu(jnp.dot(ff, fqw_ref[...],
                          preferred_element_type=jnp.float32) + fqb_ref[...])
    hidden = _gelu(jnp.dot(gfeat, w1a_ref[...],
                           preferred_element_type=jnp.float32)
                   + jnp.dot(ffeat, w1b_ref[...],
                             preferred_element_type=jnp.float32)
                   + b1_ref[...])
    o_ref[...] = jnp.dot(hidden, w2_ref[...],
                         preferred_element_type=jnp.float32) + b2_ref[...]


def _t7_call(h3, ts, Cm, Sm, ro_W, ro_b, fq_W, fq_b, W1a, W1b, b1, W2, b2):
    return pl.pallas_call(
        _t7_body,
        out_shape=jax.ShapeDtypeStruct((B, 1), jnp.float32),
    )(h3, ts, Cm, Sm, ro_W, ro_b.reshape(1, H), fq_W, fq_b.reshape(1, H),
      W1a, W1b, b1.reshape(1, H), W2, b2.reshape(1, 1))


def kernel(x, edge_index, edge_attr, batch, enc_W, enc_b, gat_W, gat_att_src,
           gat_att_dst, gat_b, gat_ln_g, gat_ln_b, gcn_W, gcn_b, gcn_ln_g,
           gcn_ln_b, ro_W, ro_b, fq_W, fq_b, ph_W1, ph_b1, ph_W2, ph_b2):
    src = edge_index[0]
    dst = edge_index[1]
    ew = edge_attr[:, 0]

    order = jnp.argsort(dst)
    dsts = jnp.take(dst, order)
    srcs = jnp.take(src, order)
    ews = jnp.take(ew, order)
    bounds = jnp.arange(NSEG + 1, dtype=jnp.int32) * NP
    offs = jnp.zeros((80,), jnp.int32).at[: NSEG + 1].set(
        jnp.searchsorted(dsts, bounds).astype(jnp.int32))

    idx64 = np.arange(H)
    As = jnp.zeros((H, 8), jnp.float32).at[idx64, idx64 // 8].set(
        gat_att_src.reshape(-1))
    Ad = jnp.zeros((H, 8), jnp.float32).at[idx64, idx64 // 8].set(
        gat_att_dst.reshape(-1))
    R = jnp.zeros((8, H), jnp.float32).at[idx64 // 8, idx64].set(1.0)
    nk = np.arange(WIN)[:, None] * np.arange(FREQ)[None, :]
    ang = 2.0 * np.pi * nk / WIN
    Cm = jnp.asarray(np.cos(ang), dtype=jnp.float32)
    Sm = jnp.asarray(np.sin(ang), dtype=jnp.float32)

    xp = jnp.pad(x, ((0, NPAD - N), (0, 0)))

    hp_g, a_s, a_d = _t1_call(xp, enc_W, enc_b, gat_W, As, Ad)

    deg_c = _deg_call(dsts, ews, offs)
    deg_flat = deg_c.reshape(NPAD)
    dinv = _t6_call(deg_flat).reshape(NPAD)
    norms = _norm_call(srcs, dsts, ews, dinv)

    a_d_c = a_d.reshape(NSEG, NP * 8)
    m_e = _gata_call(a_s, a_d_c, srcs, dsts, offs)
    m = _t2_call(m_e.reshape(NPAD, 8), a_s, a_d)
    den_c, num_c = _gatb_call(hp_g, a_s, a_d_c, m.reshape(NSEG, NP * 8),
                              srcs, dsts, offs)
    h = _t3_call(num_c.reshape(NPAD, H), den_c.reshape(NPAD, 8), m,
                 a_s, a_d, hp_g, R, gat_b, gat_ln_g, gat_ln_b)

    deg_col = deg_flat.reshape(NPAD, 1)
    for blk in range(4):
        res = h
        hp1 = _t4_call(h, gcn_W[2 * blk])
        S1 = _gcn_call(hp1, srcs, dsts, norms, offs).reshape(NPAD, H)
        o1 = _t5_call(S1, hp1, deg_col, gcn_b[2 * blk],
                      gcn_ln_g[2 * blk], gcn_ln_b[2 * blk])
        hp2 = _t4_call(o1, gcn_W[2 * blk + 1])
        S2 = _gcn_call(hp2, srcs, dsts, norms, offs).reshape(NPAD, H)
        h = _t5_call(S2, hp2, deg_col, gcn_b[2 * blk + 1],
                     gcn_ln_g[2 * blk + 1], gcn_ln_b[2 * blk + 1], res=res)

    h3 = h[:N].reshape(B, WIN, H)
    ts = x[:, 0].reshape(B, WIN)
    out = _t7_call(h3, ts, Cm, Sm, ro_W, ro_b, fq_W, fq_b,
                   ph_W1[:H], ph_W1[H:], ph_b1, ph_W2, ph_b2)
    return out


# speedup vs baseline: 16.5802x; 1.3316x over previous
import functools

import numpy as np
import jax
import jax.numpy as jnp
from jax import lax
from jax.experimental import pallas as pl
from jax.experimental.pallas import tpu as pltpu
from jax.experimental.pallas import tpu_sc as plsc

N = 50000
E = 800000
H = 64
HEADS = 8
HD = H // HEADS
WIN = 50
B = N // WIN
FREQ = WIN // 2 + 1

NSEG = 64
NP = 784
NPAD = NSEG * NP
C = 128
NCHUNK = E // C

_mesh = plsc.VectorSubcoreMesh(core_axis_name="c", subcore_axis_name="s")
_sc_params = pltpu.CompilerParams(needs_layout_passes=False, use_tc_tiling_on_sc=False)


def _wid():
    return lax.axis_index("s") * 2 + lax.axis_index("c")


def _seg_bounds(off_v, seg):
    v = off_v[pl.ds(seg, 16)]
    return v[0], v[1]


def _deg_body(dsts, ews, offs, deg_out, off_v, dv, wv, acc, cm):
    iota = lax.iota(jnp.int32, 16)
    sel0 = iota == 0
    zero16 = jnp.zeros((16,), jnp.float32)
    pltpu.sync_copy(offs, off_v)
    wid = _wid()
    for k2 in range(2):
        seg = wid * 2 + k2
        base = seg * NP
        e_lo, e_hi = _seg_bounds(off_v, seg)

        def zero_body(i, _):
            acc[pl.ds(i * 16, 16)] = zero16
            return 0
        lax.fori_loop(0, NP, zero_body, 0)

        def chunk(c, _):
            pltpu.sync_copy(dsts.at[pl.ds(c * C, C)], dv)
            pltpu.sync_copy(ews.at[pl.ds(c * C, C)], wv)
            k_lo = jnp.maximum(e_lo, c * C)
            k_hi = jnp.minimum(e_hi, (c + 1) * C)

            def edge(k, _):
                koff = k - c * C
                dl = dv[pl.ds(koff, 16)][0] - base
                w = wv[pl.ds(koff, 16)][0]
                plsc.addupdate(acc.at[pl.ds(dl * 16, 16)],
                               jnp.where(sel0, w, 0.0))
                return 0
            lax.fori_loop(k_lo, k_hi, edge, 0)
            return 0
        lax.fori_loop(e_lo // C, (e_hi + C - 1) // C, chunk, 0)

        def cp_body(i, _):
            idx = i * 256 + iota * 16
            dv_f = plsc.load_gather(acc, [idx])
            cm[pl.ds(i * 16, 16)] = dv_f + 1.0
            return 0
        lax.fori_loop(0, NP // 16, cp_body, 0)
        pltpu.sync_copy(cm, deg_out.at[seg])


def _deg_call(dsts, ews, offs):
    return pl.kernel(
        _deg_body,
        out_type=jax.ShapeDtypeStruct((NSEG, NP), jnp.float32),
        mesh=_mesh,
        compiler_params=_sc_params,
        scratch_types=[
            pltpu.VMEM((80,), jnp.int32),
            pltpu.VMEM((C,), jnp.int32),
            pltpu.VMEM((C,), jnp.float32),
            pltpu.VMEM((NP * 16,), jnp.float32),
            pltpu.VMEM((NP,), jnp.float32),
        ],
    )(dsts, ews, offs)


def _norm_body(srcs, dsts, ews, dinv, norm_out, dinv_v, sv, dv, wv, nv):
    wid = _wid()
    pltpu.sync_copy(dinv, dinv_v)
    c0 = wid * NCHUNK // 32
    c1 = (wid + 1) * NCHUNK // 32

    def chunk(c, _):
        pltpu.sync_copy(srcs.at[pl.ds(c * C, C)], sv)
        pltpu.sync_copy(dsts.at[pl.ds(c * C, C)], dv)
        pltpu.sync_copy(ews.at[pl.ds(c * C, C)], wv)
        for u in range(C // 16):
            s16 = sv[pl.ds(u * 16, 16)]
            d16 = dv[pl.ds(u * 16, 16)]
            w16 = wv[pl.ds(u * 16, 16)]
            n16 = (plsc.load_gather(dinv_v, [s16]) * w16
                   * plsc.load_gather(dinv_v, [d16]))
            nv[pl.ds(u * 16, 16)] = n16
        pltpu.sync_copy(nv, norm_out.at[pl.ds(c * C, C)])
        return 0
    lax.fori_loop(c0, c1, chunk, 0)


def _norm_call(srcs, dsts, ews, dinv):
    return pl.kernel(
        _norm_body,
        out_type=jax.ShapeDtypeStruct((E,), jnp.float32),
        mesh=_mesh,
        compiler_params=_sc_params,
        scratch_types=[
            pltpu.VMEM((NPAD,), jnp.float32),
            pltpu.VMEM((C,), jnp.int32),
            pltpu.VMEM((C,), jnp.int32),
            pltpu.VMEM((C,), jnp.float32),
            pltpu.VMEM((C,), jnp.float32),
        ],
    )(srcs, dsts, ews, dinv)


def _gcn_body(hp, srcs, dsts, norms, offs, out,
              off_v, sv, dv, nv, rows, acc, sem):
    iota = lax.iota(jnp.int32, 16)
    zero16 = jnp.zeros((16,), jnp.float32)
    pltpu.sync_copy(offs, off_v)
    wid = _wid()
    for k2 in range(2):
        seg = wid * 2 + k2
        base = seg * NP
        e_lo, e_hi = _seg_bounds(off_v, seg)

        def zero_body(i, _):
            for j in range(4):
                acc[pl.ds(i * 64 + j * 16, 16)] = zero16
            return 0
        lax.fori_loop(0, NP, zero_body, 0)

        def chunk(c, _):
            pltpu.sync_copy(srcs.at[pl.ds(c * C, C)], sv)
            pltpu.sync_copy(dsts.at[pl.ds(c * C, C)], dv)
            pltpu.sync_copy(norms.at[pl.ds(c * C, C)], nv.at[pl.ds(0, C)])
            pltpu.async_copy(hp.at[sv], rows, sem).wait()
            g_lo = e_lo - c * C
            g_hi = e_hi - c * C
            for u in range(8):
                lidx = iota + u * 16
                valid = (lidx >= g_lo) & (lidx < g_hi)
                n16 = jnp.where(valid, nv[pl.ds(u * 16, 16)], 0.0)
                dl16 = jnp.clip(dv[pl.ds(u * 16, 16)] - base, 0, NP - 1)
                nv[pl.ds(u * 16, 16)] = n16
                for bq in range(4):
                    prods = []
                    for i in range(4):
                        ii = u * 16 + bq * 4 + i
                        w = nv[pl.ds(ii, 16)][0]
                        for j in range(4):
                            prods.append(w * rows[ii, pl.ds(j * 16, 16)])
                    for i in range(4):
                        dl = dl16[bq * 4 + i]
                        for j in range(4):
                            plsc.addupdate(
                                acc.at[pl.ds(dl * 64 + j * 16, 16)],
                                prods[i * 4 + j])
            return 0
        lax.fori_loop(e_lo // C, (e_hi + C - 1) // C, chunk, 0)
        pltpu.sync_copy(acc, out.at[seg])


def _gcn_call(hp, srcs, dsts, norms, offs):
    return pl.kernel(
        _gcn_body,
        out_type=jax.ShapeDtypeStruct((NSEG, NP * 64), jnp.float32),
        mesh=_mesh,
        compiler_params=_sc_params,
        scratch_types=[
            pltpu.VMEM((80,), jnp.int32),
            pltpu.VMEM((C,), jnp.int32),
            pltpu.VMEM((C,), jnp.int32),
            pltpu.VMEM((C + 16,), jnp.float32),
            pltpu.VMEM((C, 64), jnp.float32),
            pltpu.VMEM((NP * 64,), jnp.float32),
            pltpu.SemaphoreType.DMA,
        ],
    )(hp, srcs, dsts, norms, offs)


def _gata_body(a_s, a_d_c, srcs, dsts, offs, m_out,
               off_v, sv, dv, as_ch, adv, acc_m, cmp_v, sem):
    iota = lax.iota(jnp.int32, 16)
    rsel = jnp.where(iota >= 8, 1, 0)
    ci = iota % 8
    pairsel = (iota // 8) * 16 + (iota % 8)
    neg = jnp.full((16,), -1e30, jnp.float32)
    pltpu.sync_copy(offs, off_v)
    wid = _wid()
    for k2 in range(2):
        seg = wid * 2 + k2
        base = seg * NP
        e_lo, e_hi = _seg_bounds(off_v, seg)
        pltpu.sync_copy(a_d_c.at[seg], adv.at[pl.ds(0, NP * 8)])

        def init_body(i, _):
            acc_m[pl.ds(i * 16, 16)] = neg
            return 0
        lax.fori_loop(0, NP, init_body, 0)

        def chunk(c, _):
            pltpu.sync_copy(srcs.at[pl.ds(c * C, C)], sv)
            pltpu.sync_copy(dsts.at[pl.ds(c * C, C)], dv)
            pltpu.async_copy(a_s.at[sv], as_ch.at[pl.ds(0, C)], sem).wait()
            k_lo = jnp.maximum(e_lo, c * C)
            k_hi = jnp.minimum(e_hi, (c + 1) * C)

            def edge(k, _):
                koff = k - c * C
                dl = dv[pl.ds(koff, 16)][0] - base
                as16 = plsc.load_gather(as_ch, [koff + rsel, ci])
                ad16 = adv[pl.ds(dl * 8, 16)]
                e16 = as16 + ad16
                e16 = jnp.where(e16 > 0, e16, 0.2 * e16)
                old = acc_m[pl.ds(dl * 16, 16)]
                acc_m[pl.ds(dl * 16, 16)] = jnp.maximum(old, e16)
                return 0
            lax.fori_loop(k_lo, k_hi, edge, 0)
            return 0
        lax.fori_loop(e_lo // C, (e_hi + C - 1) // C, chunk, 0)

        def cp_body(i, _):
            cmp_v[pl.ds(i * 16, 16)] = plsc.load_gather(acc_m, [i * 32 + pairsel])
            return 0
        lax.fori_loop(0, NP * 8 // 16, cp_body, 0)
        pltpu.sync_copy(cmp_v, m_out.at[seg])


def _gata_call(a_s, a_d_c, srcs, dsts, offs):
    return pl.kernel(
        _gata_body,
        out_type=jax.ShapeDtypeStruct((NSEG, NP * 8), jnp.float32),
        mesh=_mesh,
        compiler_params=_sc_params,
        scratch_types=[
            pltpu.VMEM((80,), jnp.int32),
            pltpu.VMEM((C,), jnp.int32),
            pltpu.VMEM((C,), jnp.int32),
            pltpu.VMEM((C + 2, 8), jnp.float32),
            pltpu.VMEM((NP * 8 + 16,), jnp.float32),
            pltpu.VMEM((NP * 16,), jnp.float32),
            pltpu.VMEM((NP * 8,), jnp.float32),
            pltpu.SemaphoreType.DMA,
        ],
    )(a_s, a_d_c, srcs, dsts, offs)


def _gatb_body(hp, a_s, a_d_c, m_c, srcs, dsts, offs, den_out, num_out,
               off_v, sv, dv, as_ch, adv, mv, rows, exv, acc_den, acc_num,
               cmp_v, sem):
    iota = lax.iota(jnp.int32, 16)
    rsel = jnp.where(iota >= 8, 1, 0)
    ci = iota % 8
    pairsel = (iota // 8) * 16 + (iota % 8)
    pats = [2 * j + rsel for j in range(4)]
    zero16 = jnp.zeros((16,), jnp.float32)
    pltpu.sync_copy(offs, off_v)
    wid = _wid()
    for k2 in range(2):
        seg = wid * 2 + k2
        base = seg * NP
        e_lo, e_hi = _seg_bounds(off_v, seg)
        pltpu.sync_copy(a_d_c.at[seg], adv.at[pl.ds(0, NP * 8)])
        pltpu.sync_copy(m_c.at[seg], mv.at[pl.ds(0, NP * 8)])

        def init_body(i, _):
            acc_den[pl.ds(i * 16, 16)] = zero16
            for j in range(4):
                acc_num[pl.ds(i * 64 + j * 16, 16)] = zero16
            return 0
        lax.fori_loop(0, NP, init_body, 0)

        def chunk(c, _):
            pltpu.sync_copy(srcs.at[pl.ds(c * C, C)], sv)
            pltpu.sync_copy(dsts.at[pl.ds(c * C, C)], dv)
            pltpu.async_copy(a_s.at[sv], as_ch.at[pl.ds(0, C)], sem).wait()
            pltpu.async_copy(hp.at[sv], rows, sem).wait()
            k_lo = jnp.maximum(e_lo, c * C)
            k_hi = jnp.minimum(e_hi, (c + 1) * C)

            def edge(k, _):
                koff = k - c * C
                dl = dv[pl.ds(koff, 16)][0] - base
                as16 = plsc.load_gather(as_ch, [koff + rsel, ci])
                ad16 = adv[pl.ds(dl * 8, 16)]
                e16 = as16 + ad16
                e16 = jnp.where(e16 > 0, e16, 0.2 * e16)
                m16 = mv[pl.ds(dl * 8, 16)]
                ex16 = jnp.exp(e16 - m16)
                plsc.addupdate(acc_den.at[pl.ds(dl * 16, 16)], ex16)
                exv[pl.ds(0, 16)] = ex16
                for j in range(4):
                    exj = plsc.load_gather(exv, [pats[j]])
                    plsc.addupdate(acc_num.at[pl.ds(dl * 64 + j * 16, 16)],
                                   exj * rows[koff, pl.ds(j * 16, 16)])
                return 0
            lax.fori_loop(k_lo, k_hi, edge, 0)
            return 0
        lax.fori_loop(e_lo // C, (e_hi + C - 1) // C, chunk, 0)

        def cp_body(i, _):
            cmp_v[pl.ds(i * 16, 16)] = plsc.load_gather(acc_den, [i * 32 + pairsel])
            return 0
        lax.fori_loop(0, NP * 8 // 16, cp_body, 0)
        pltpu.sync_copy(cmp_v, den_out.at[seg])
        pltpu.sync_copy(acc_num, num_out.at[seg])


def _gatb_call(hp, a_s, a_d_c, m_c, srcs, dsts, offs):
    return pl.kernel(
        _gatb_body,
        out_type=(jax.ShapeDtypeStruct((NSEG, NP * 8), jnp.float32),
                  jax.ShapeDtypeStruct((NSEG, NP * 64), jnp.float32)),
        mesh=_mesh,
        compiler_params=_sc_params,
        scratch_types=[
            pltpu.VMEM((80,), jnp.int32),
            pltpu.VMEM((C,), jnp.int32),
            pltpu.VMEM((C,), jnp.int32),
            pltpu.VMEM((C + 2, 8), jnp.float32),
            pltpu.VMEM((NP * 8 + 16,), jnp.float32),
            pltpu.VMEM((NP * 8 + 16,), jnp.float32),
            pltpu.VMEM((C, 64), jnp.float32),
            pltpu.VMEM((16,), jnp.float32),
            pltpu.VMEM((NP * 16,), jnp.float32),
            pltpu.VMEM((NP * 64,), jnp.float32),
            pltpu.VMEM((NP * 8,), jnp.float32),
            pltpu.SemaphoreType.DMA,
        ],
    )(hp, a_s, a_d_c, m_c, srcs, dsts, offs)


def _gelu(x):
    return 0.5 * x * (1.0 + lax.erf(x * 0.7071067811865476))


def _ln(x, g, b):
    mu = jnp.mean(x, axis=-1, keepdims=True)
    v = jnp.mean((x - mu) ** 2, axis=-1, keepdims=True)
    return (x - mu) / jnp.sqrt(v + 1e-5) * g + b


ROWB = 512
GRID = NPAD // ROWB


def _t1_body(x_ref, ew_ref, eb_ref, gw_ref, as_ref, ad_ref,
             hp_ref, s_ref, d_ref):
    h0 = _gelu(jnp.dot(x_ref[...], ew_ref[...],
                       preferred_element_type=jnp.float32) + eb_ref[...])
    hp = jnp.dot(h0, gw_ref[...], preferred_element_type=jnp.float32)
    hp_ref[...] = hp
    s_ref[...] = jnp.dot(hp, as_ref[...], preferred_element_type=jnp.float32)
    d_ref[...] = jnp.dot(hp, ad_ref[...], preferred_element_type=jnp.float32)


def _t1_call(xp, enc_W, enc_b, gat_W, As, Ad):
    return pl.pallas_call(
        _t1_body,
        grid=(GRID,),
        in_specs=[
            pl.BlockSpec((ROWB, 4), lambda i: (i, 0)),
            pl.BlockSpec((4, H), lambda i: (0, 0)),
            pl.BlockSpec((1, H), lambda i: (0, 0)),
            pl.BlockSpec((H, H), lambda i: (0, 0)),
            pl.BlockSpec((H, 8), lambda i: (0, 0)),
            pl.BlockSpec((H, 8), lambda i: (0, 0)),
        ],
        out_specs=[
            pl.BlockSpec((ROWB, H), lambda i: (i, 0)),
            pl.BlockSpec((ROWB, 8), lambda i: (i, 0)),
            pl.BlockSpec((ROWB, 8), lambda i: (i, 0)),
        ],
        out_shape=[
            jax.ShapeDtypeStruct((NPAD, H), jnp.float32),
            jax.ShapeDtypeStruct((NPAD, 8), jnp.float32),
            jax.ShapeDtypeStruct((NPAD, 8), jnp.float32),
        ],
    )(xp, enc_W, enc_b.reshape(1, H), gat_W, As, Ad)


def _t6_body(deg_ref, dinv_ref):
    dinv_ref[...] = lax.rsqrt(deg_ref[...])


def _t6_call(deg):
    return pl.pallas_call(
        _t6_body,
        out_shape=jax.ShapeDtypeStruct((392, 128), jnp.float32),
    )(deg.reshape(392, 128))


def _t2_body(me_ref, as_ref, ad_ref, m_ref):
    es = as_ref[...] + ad_ref[...]
    es = jnp.where(es > 0, es, 0.2 * es)
    m_ref[...] = jnp.maximum(me_ref[...], es)


def _t2_call(m_e, a_s, a_d):
    return pl.pallas_call(
        _t2_body,
        grid=(GRID,),
        in_specs=[pl.BlockSpec((ROWB, 8), lambda i: (i, 0))] * 3,
        out_specs=pl.BlockSpec((ROWB, 8), lambda i: (i, 0)),
        out_shape=jax.ShapeDtypeStruct((NPAD, 8), jnp.float32),
    )(m_e, a_s, a_d)


def _t3_body(num_ref, den_ref, m_ref, as_ref, ad_ref, hp_ref, r_ref,
             b_ref, g_ref, lb_ref, h_ref):
    es = as_ref[...] + ad_ref[...]
    es = jnp.where(es > 0, es, 0.2 * es)
    w = jnp.exp(es - m_ref[...])
    den = den_ref[...] + w + 1e-16
    w64 = jnp.dot(w, r_ref[...], preferred_element_type=jnp.float32)
    den64 = jnp.dot(den, r_ref[...], preferred_element_type=jnp.float32)
    out = (num_ref[...] + w64 * hp_ref[...]) / den64 + b_ref[...]
    h_ref[...] = _ln(out, g_ref[...], lb_ref[...])


def _t3_call(num, den, m, a_s, a_d, hp, R, gat_b, ln_g, ln_b):
    return pl.pallas_call(
        _t3_body,
        grid=(GRID,),
        in_specs=[
            pl.BlockSpec((ROWB, H), lambda i: (i, 0)),
            pl.BlockSpec((ROWB, 8), lambda i: (i, 0)),
            pl.BlockSpec((ROWB, 8), lambda i: (i, 0)),
            pl.BlockSpec((ROWB, 8), lambda i: (i, 0)),
            pl.BlockSpec((ROWB, 8), lambda i: (i, 0)),
            pl.BlockSpec((ROWB, H), lambda i: (i, 0)),
            pl.BlockSpec((8, H), lambda i: (0, 0)),
            pl.BlockSpec((1, H), lambda i: (0, 0)),
            pl.BlockSpec((1, H), lambda i: (0, 0)),
            pl.BlockSpec((1, H), lambda i: (0, 0)),
        ],
        out_specs=pl.BlockSpec((ROWB, H), lambda i: (i, 0)),
        out_shape=jax.ShapeDtypeStruct((NPAD, H), jnp.float32),
    )(num, den, m, a_s, a_d, hp, R, gat_b.reshape(1, H),
      ln_g.reshape(1, H), ln_b.reshape(1, H))


def _t4_body(h_ref, w_ref, o_ref):
    o_ref[...] = jnp.dot(h_ref[...], w_ref[...],
                         preferred_element_type=jnp.float32)


def _t4_call(h, W):
    return pl.pallas_call(
        _t4_body,
        grid=(GRID,),
        in_specs=[
            pl.BlockSpec((ROWB, H), lambda i: (i, 0)),
            pl.BlockSpec((H, H), lambda i: (0, 0)),
        ],
        out_specs=pl.BlockSpec((ROWB, H), lambda i: (i, 0)),
        out_shape=jax.ShapeDtypeStruct((NPAD, H), jnp.float32),
    )(h, W)


def _t5_body(s_ref, hp_ref, deg_ref, b_ref, g_ref, lb_ref, res_ref, o_ref):
    t = s_ref[...] + hp_ref[...] / deg_ref[...] + b_ref[...]
    o = _gelu(_ln(t, g_ref[...], lb_ref[...]))
    if res_ref is not None:
        o = o + res_ref[...]
    o_ref[...] = o


def _t5_call(S, hp, deg_col, b, g, lb, res=None):
    ins = [S, hp, deg_col, b.reshape(1, H), g.reshape(1, H), lb.reshape(1, H)]
    specs = [
        pl.BlockSpec((ROWB, H), lambda i: (i, 0)),
        pl.BlockSpec((ROWB, H), lambda i: (i, 0)),
        pl.BlockSpec((ROWB, 1), lambda i: (i, 0)),
        pl.BlockSpec((1, H), lambda i: (0, 0)),
        pl.BlockSpec((1, H), lambda i: (0, 0)),
        pl.BlockSpec((1, H), lambda i: (0, 0)),
    ]
    if res is not None:
        ins.append(res)
        specs.append(pl.BlockSpec((ROWB, H), lambda i: (i, 0)))
        body = _t5_body
    else:
        def body(s_ref, hp_ref, deg_ref, b_ref, g_ref, lb_ref, o_ref):
            _t5_body(s_ref, hp_ref, deg_ref, b_ref, g_ref, lb_ref, None, o_ref)
    return pl.pallas_call(
        body,
        grid=(GRID,),
        in_specs=specs,
        out_specs=pl.BlockSpec((ROWB, H), lambda i: (i, 0)),
        out_shape=jax.ShapeDtypeStruct((NPAD, H), jnp.float32),
    )(*ins)


def _t7_body(h3_ref, ts_ref, cm_ref, sm_ref, row_ref, rob_ref,
             fqw_ref, fqb_ref, w1a_ref, w1b_ref, b1_ref, w2_ref, b2_ref,
             o_ref):
    gsum = jnp.sum(h3_ref[...], axis=1)
    gemb = gsum * (1.0 / WIN)
    gfeat = _gelu(jnp.dot(gemb, row_ref[...],
                          preferred_element_type=jnp.float32) + rob_ref[...])
    ts = ts_ref[...]
    fr = jnp.dot(ts, cm_ref[...], preferred_element_type=jnp.float32)
    fi = jnp.dot(ts, sm_ref[...], preferred_element_type=jnp.float32)
    ff = jnp.sqrt(fr * fr + fi * fi)
    ffeat = _gelu(jnp.dot(ff, fqw_ref[...],
                          preferred_element_type=jnp.float32) + fqb_ref[...])
    hidden = _gelu(jnp.dot(gfeat, w1a_ref[...],
                           preferred_element_type=jnp.float32)
                   + jnp.dot(ffeat, w1b_ref[...],
                             preferred_element_type=jnp.float32)
                   + b1_ref[...])
    o_ref[...] = jnp.dot(hidden, w2_ref[...],
                         preferred_element_type=jnp.float32) + b2_ref[...]


def _t7_call(h3, ts, Cm, Sm, ro_W, ro_b, fq_W, fq_b, W1a, W1b, b1, W2, b2):
    return pl.pallas_call(
        _t7_body,
        out_shape=jax.ShapeDtypeStruct((B, 1), jnp.float32),
    )(h3, ts, Cm, Sm, ro_W, ro_b.reshape(1, H), fq_W, fq_b.reshape(1, H),
      W1a, W1b, b1.reshape(1, H), W2, b2.reshape(1, 1))


def kernel(x, edge_index, edge_attr, batch, enc_W, enc_b, gat_W, gat_att_src,
           gat_att_dst, gat_b, gat_ln_g, gat_ln_b, gcn_W, gcn_b, gcn_ln_g,
           gcn_ln_b, ro_W, ro_b, fq_W, fq_b, ph_W1, ph_b1, ph_W2, ph_b2):
    src = edge_index[0]
    dst = edge_index[1]
    ew = edge_attr[:, 0]

    order = jnp.argsort(dst)
    dsts = jnp.take(dst, order)
    srcs = jnp.take(src, order)
    ews = jnp.take(ew, order)
    bounds = jnp.arange(NSEG + 1, dtype=jnp.int32) * NP
    offs = jnp.zeros((80,), jnp.int32).at[: NSEG + 1].set(
        jnp.searchsorted(dsts, bounds).astype(jnp.int32))

    idx64 = np.arange(H)
    As = jnp.zeros((H, 8), jnp.float32).at[idx64, idx64 // 8].set(
        gat_att_src.reshape(-1))
    Ad = jnp.zeros((H, 8), jnp.float32).at[idx64, idx64 // 8].set(
        gat_att_dst.reshape(-1))
    R = jnp.zeros((8, H), jnp.float32).at[idx64 // 8, idx64].set(1.0)
    nk = np.arange(WIN)[:, None] * np.arange(FREQ)[None, :]
    ang = 2.0 * np.pi * nk / WIN
    Cm = jnp.asarray(np.cos(ang), dtype=jnp.float32)
    Sm = jnp.asarray(np.sin(ang), dtype=jnp.float32)

    xp = jnp.pad(x, ((0, NPAD - N), (0, 0)))

    hp_g, a_s, a_d = _t1_call(xp, enc_W, enc_b, gat_W, As, Ad)

    deg_c = _deg_call(dsts, ews, offs)
    deg_flat = deg_c.reshape(NPAD)
    dinv = _t6_call(deg_flat).reshape(NPAD)
    norms = _norm_call(srcs, dsts, ews, dinv)

    a_d_c = a_d.reshape(NSEG, NP * 8)
    m_e = _gata_call(a_s, a_d_c, srcs, dsts, offs)
    m = _t2_call(m_e.reshape(NPAD, 8), a_s, a_d)
    den_c, num_c = _gatb_call(hp_g, a_s, a_d_c, m.reshape(NSEG, NP * 8),
                              srcs, dsts, offs)
    h = _t3_call(num_c.reshape(NPAD, H), den_c.reshape(NPAD, 8), m,
                 a_s, a_d, hp_g, R, gat_b, gat_ln_g, gat_ln_b)

    deg_col = deg_flat.reshape(NPAD, 1)
    for blk in range(4):
        res = h
        hp1 = _t4_call(h, gcn_W[2 * blk])
        S1 = _gcn_call(hp1, srcs, dsts, norms, offs).reshape(NPAD, H)
        o1 = _t5_call(S1, hp1, deg_col, gcn_b[2 * blk],
                      gcn_ln_g[2 * blk], gcn_ln_b[2 * blk])
        hp2 = _t4_call(o1, gcn_W[2 * blk + 1])
        S2 = _gcn_call(hp2, srcs, dsts, norms, offs).reshape(NPAD, H)
        h = _t5_call(S2, hp2, deg_col, gcn_b[2 * blk + 1],
                     gcn_ln_g[2 * blk + 1], gcn_ln_b[2 * blk + 1], res=res)

    h3 = h[:N].reshape(B, WIN, H)
    ts = x[:, 0].reshape(B, WIN)
    out = _t7_call(h3, ts, Cm, Sm, ro_W, ro_b, fq_W, fq_b,
                   ph_W1[:H], ph_W1[H:], ph_b1, ph_W2, ph_b2)
    return out
